# Initial kernel scaffold; baseline (speedup 1.0000x reference)
import jax
import jax.numpy as jnp
from jax.experimental import pallas as pl


def kernel(x, tables):
    raise NotImplementedError("write your pallas kernel here")


# speedup vs baseline: 2.3432x; 2.3432x over previous
import functools

import jax
import jax.numpy as jnp
from jax import lax
from jax.experimental import pallas as pl
from jax.experimental.pallas import tpu as pltpu
from jax.experimental.pallas import tpu_sc as plsc

_N_TABLES = 26
_D = 32
_NC = 2
_NS = 16
_NW = _NC * _NS


def _body(x_hbm, tables_hbm, out_hbm, idx_v, rows_a, rows_b, gsem):
    bpw = x_hbm.shape[0] // _NW
    wid = lax.axis_index("s") * _NC + lax.axis_index("c")
    base = wid * bpw
    pltpu.sync_copy(x_hbm.at[pl.ds(base, bpw)], idx_v)
    bufs = (rows_a, rows_b)
    cp = pltpu.async_copy(tables_hbm.at[0].at[idx_v], bufs[0], gsem)
    for t in range(_N_TABLES):
        if t + 1 < _N_TABLES:
            nxt = pltpu.async_copy(
                tables_hbm.at[t + 1].at[idx_v], bufs[(t + 1) % 2], gsem)
        cp.wait()
        pltpu.sync_copy(bufs[t % 2], out_hbm.at[pl.ds(base, bpw), t])
        if t + 1 < _N_TABLES:
            cp = nxt


def kernel(x, tables):
    b = x.shape[0]
    bpw = b // _NW
    run = pl.kernel(
        _body,
        out_type=jax.ShapeDtypeStruct((b, _N_TABLES, _D), jnp.float32),
        mesh=plsc.VectorSubcoreMesh(
            core_axis_name="c", subcore_axis_name="s",
            num_cores=_NC, num_subcores=_NS),
        scratch_types=[
            pltpu.VMEM((bpw,), jnp.int32),
            pltpu.VMEM((bpw, _D), jnp.float32),
            pltpu.VMEM((bpw, _D), jnp.float32),
            pltpu.SemaphoreType.DMA,
        ],
        compiler_params=pltpu.CompilerParams(use_tc_tiling_on_sc=False),
    )
    return run(x.astype(jnp.int32), tables)
